# baseline (speedup 1.0000x reference)
import functools

import jax
import jax.numpy as jnp
from jax import lax
from jax.experimental import pallas as pl
from jax.experimental.pallas import tpu as pltpu
from jax.experimental.pallas import tpu_sc as plsc

_N = 10000
_E = 320000
_H = 128
_G = 64
_NL = 3
_NC = 2
_NS = 16
_NW = _NC * _NS
_CW = 128
_NCH = 80
_EPT = _CW * _NCH
_EPAD = _EPT * _NW
_APAD = 10112
_RPT = _APAD // _NS
_HCH = _NCH // 2


def _sc_agg(h_pad, srcr, dstr, zrows):
    mesh = plsc.VectorSubcoreMesh(core_axis_name="c", subcore_axis_name="s")

    @functools.partial(
        pl.kernel,
        out_type=jax.ShapeDtypeStruct((_NC, _APAD, _H), jnp.float32),
        mesh=mesh,
        scratch_types=[
            pltpu.VMEM((_HCH, _CW), jnp.int32),
            pltpu.VMEM((_HCH, _CW), jnp.int32),
            pltpu.VMEM((_CW, _H), jnp.float32),
            pltpu.VMEM((_CW, _H), jnp.float32),
            pltpu.VMEM_SHARED((_APAD, _H), jnp.float32),
            pltpu.SemaphoreType.DMA,
        ],
    )
    def agg(h_hbm, src_hbm, dst_hbm, z_hbm, out_hbm, src_v, dst_v, rows_a,
            rows_b, acc, sem):
        c = lax.axis_index("c")
        s = lax.axis_index("s")
        wid = s * _NC + c
        r0 = s * _RPT
        pltpu.sync_copy(z_hbm, acc.at[pl.ds(r0, _RPT)])
        plsc.subcore_barrier()

        for half in range(2):
            pltpu.sync_copy(src_hbm.at[wid, pl.ds(half * _HCH, _HCH)], src_v)
            pltpu.sync_copy(dst_hbm.at[wid, pl.ds(half * _HCH, _HCH)], dst_v)
            pltpu.async_copy(h_hbm.at[src_v.at[0]], rows_a, sem).wait()

            def step(jo, carry):
                j = 2 * jo
                cp_b = pltpu.async_copy(h_hbm.at[src_v.at[j + 1]], rows_b,
                                        sem)
                pltpu.sync_copy(rows_a, acc.at[dst_v.at[j]], add=True)
                cp_b.wait()
                j2 = jnp.minimum(j + 2, _HCH - 1)
                cp_a = pltpu.async_copy(h_hbm.at[src_v.at[j2]], rows_a, sem)
                pltpu.sync_copy(rows_b, acc.at[dst_v.at[j + 1]], add=True)
                cp_a.wait()
                return carry

            lax.fori_loop(0, _HCH // 2, step, 0)
        plsc.subcore_barrier()
        pltpu.sync_copy(acc.at[pl.ds(r0, _RPT)],
                        out_hbm.at[c, pl.ds(r0, _RPT)])

    return agg(h_pad, srcr, dstr, zrows)


def _tc_fc0(x, W, b):
    def body(x_ref, w_ref, b_ref, o_ref):
        o_ref[...] = jnp.dot(x_ref[...], w_ref[...],
                             preferred_element_type=jnp.float32) + b_ref[...]

    return pl.pallas_call(
        body,
        out_shape=jax.ShapeDtypeStruct((_N, _H), jnp.float32),
    )(x, W, b.reshape(1, _H))


def _tc_layer(h, agg2, W1, b1, a1, gin, bein, W2, b2, a2, gout, beout):
    def body(h_ref, agg_ref, w1_ref, b1_ref, a1_ref, gi_ref, bi_ref,
             w2_ref, b2_ref, a2_ref, go_ref, bo_ref, o_ref):
        m = h_ref[...] + agg_ref[0, :_N] + agg_ref[1, :_N]
        t = jnp.dot(m, w1_ref[...],
                    preferred_element_type=jnp.float32) + b1_ref[...]
        t = jnp.where(t >= 0, t, a1_ref[0, 0] * t)
        mu = jnp.mean(t, axis=0, keepdims=True)
        var = jnp.mean((t - mu) ** 2, axis=0, keepdims=True)
        t = gi_ref[...] * (t - mu) / jnp.sqrt(var + 1e-5) + bi_ref[...]
        t = jnp.dot(t, w2_ref[...],
                    preferred_element_type=jnp.float32) + b2_ref[...]
        t = jnp.where(t >= 0, t, a2_ref[0, 0] * t)
        mu2 = jnp.mean(t, axis=0, keepdims=True)
        var2 = jnp.mean((t - mu2) ** 2, axis=0, keepdims=True)
        o_ref[...] = (go_ref[...] * (t - mu2) / jnp.sqrt(var2 + 1e-5)
                      + bo_ref[...])

    return pl.pallas_call(
        body,
        out_shape=jax.ShapeDtypeStruct((_N, _H), jnp.float32),
    )(h, agg2, W1, b1.reshape(1, _H), a1.reshape(1, 1), gin.reshape(1, _H),
      bein.reshape(1, _H), W2, b2.reshape(1, _H), a2.reshape(1, 1),
      gout.reshape(1, _H), beout.reshape(1, _H))


def _tc_final(h, batch2d, bn_g, bn_b, fc_W, fc_b):
    def body(h_ref, b_ref, g_ref, bb_ref, w_ref, fb_ref, o_ref):
        gids = lax.broadcasted_iota(jnp.int32, (_G, _N), 0)
        oh = (b_ref[...] == gids).astype(jnp.float32)
        pooled = jnp.dot(oh, h_ref[...], preferred_element_type=jnp.float32)
        mu = jnp.mean(pooled, axis=0, keepdims=True)
        var = jnp.mean((pooled - mu) ** 2, axis=0, keepdims=True)
        z = g_ref[...] * (pooled - mu) / jnp.sqrt(var + 1e-5) + bb_ref[...]
        o_ref[...] = jnp.dot(z, w_ref[...],
                             preferred_element_type=jnp.float32) + fb_ref[...]

    lat = fc_W.shape[1]
    return pl.pallas_call(
        body,
        out_shape=jax.ShapeDtypeStruct((_G, lat), jnp.float32),
    )(h, batch2d, bn_g.reshape(1, _H), bn_b.reshape(1, _H), fc_W,
      fc_b.reshape(1, lat))


def kernel(x, edge_index, batch, fc0_W, fc0_b, W1, b1, a1, g_in, be_in,
           W2, b2, a2, g_out, be_out, bn_g, bn_b, fc_W, fc_b):
    order = jnp.argsort(edge_index[0])
    src = edge_index[0][order]
    dst = edge_index[1][order]
    pad = _EPAD - _E
    srcr = jnp.concatenate(
        [src, jnp.full((pad,), _N, src.dtype)]).reshape(_NW, _NCH, _CW)
    dstr = jnp.concatenate(
        [dst, jnp.zeros((pad,), dst.dtype)]).reshape(_NW, _NCH, _CW)
    zrows = jnp.zeros((_RPT, _H), jnp.float32)

    h = _tc_fc0(x, fc0_W, fc0_b)
    for i in range(_NL):
        hp = jnp.concatenate([h, jnp.zeros((8, _H), jnp.float32)], axis=0)
        agg2 = _sc_agg(hp, srcr, dstr, zrows)
        h = _tc_layer(h, agg2, W1[i], b1[i], a1[i].reshape(1, 1),
                      g_in[i], be_in[i], W2[i], b2[i], a2[i].reshape(1, 1),
                      g_out[i], be_out[i])
    return _tc_final(h, batch.reshape(1, _N), bn_g, bn_b, fc_W, fc_b)


# speedup vs baseline: 1.3676x; 1.3676x over previous
import functools

import jax
import jax.numpy as jnp
from jax import lax
from jax.experimental import pallas as pl
from jax.experimental.pallas import tpu as pltpu
from jax.experimental.pallas import tpu_sc as plsc

_N = 10000
_E = 320000
_H = 128
_G = 64
_NL = 3
_NC = 2
_NS = 16
_NW = _NC * _NS
_CW = 128
_NCH = 80
_EPT = _CW * _NCH
_EPAD = _EPT * _NW
_APAD = 10112
_RPT = _APAD // _NS
_HCH = _NCH // 2


def _sc_agg(h_pad, srcr, dstr, zrows):
    mesh = plsc.VectorSubcoreMesh(core_axis_name="c", subcore_axis_name="s")

    @functools.partial(
        pl.kernel,
        out_type=jax.ShapeDtypeStruct((_NC, _APAD, _H), jnp.float32),
        mesh=mesh,
        scratch_types=[
            pltpu.VMEM((_HCH, _CW), jnp.int32),
            pltpu.VMEM((_HCH, _CW), jnp.int32),
            pltpu.VMEM((_CW, _H), jnp.float32),
            pltpu.VMEM((_CW, _H), jnp.float32),
            pltpu.VMEM_SHARED((_APAD, _H), jnp.float32),
            pltpu.SemaphoreType.DMA,
        ],
    )
    def agg(h_hbm, src_hbm, dst_hbm, z_hbm, out_hbm, src_v, dst_v, rows_a,
            rows_b, acc, sem):
        c = lax.axis_index("c")
        s = lax.axis_index("s")
        wid = s * _NC + c
        r0 = s * _RPT
        pltpu.sync_copy(z_hbm, acc.at[pl.ds(r0, _RPT)])
        plsc.subcore_barrier()

        for half in range(2):
            pltpu.sync_copy(src_hbm.at[wid, pl.ds(half * _HCH, _HCH)], src_v)
            pltpu.sync_copy(dst_hbm.at[wid, pl.ds(half * _HCH, _HCH)], dst_v)
            pltpu.async_copy(h_hbm.at[src_v.at[0]], rows_a, sem).wait()

            def step(jo, carry):
                j = 2 * jo
                cp_b = pltpu.async_copy(h_hbm.at[src_v.at[j + 1]], rows_b,
                                        sem)
                pltpu.sync_copy(rows_a, acc.at[dst_v.at[j]], add=True)
                cp_b.wait()
                j2 = jnp.minimum(j + 2, _HCH - 1)
                cp_a = pltpu.async_copy(h_hbm.at[src_v.at[j2]], rows_a, sem)
                pltpu.sync_copy(rows_b, acc.at[dst_v.at[j + 1]], add=True)
                cp_a.wait()
                return carry

            lax.fori_loop(0, _HCH // 2, step, 0)
        plsc.subcore_barrier()
        pltpu.sync_copy(acc.at[pl.ds(r0, _RPT)],
                        out_hbm.at[c, pl.ds(r0, _RPT)])

    return agg(h_pad, srcr, dstr, zrows)


def _tc_fc0(x, W, b):
    def body(x_ref, w_ref, b_ref, o_ref):
        o_ref[...] = jnp.dot(x_ref[...], w_ref[...],
                             preferred_element_type=jnp.float32) + b_ref[...]

    return pl.pallas_call(
        body,
        out_shape=jax.ShapeDtypeStruct((_N, _H), jnp.float32),
    )(x, W, b.reshape(1, _H))


def _tc_layer(h, agg2, W1, b1, a1, gin, bein, W2, b2, a2, gout, beout):
    def body(h_ref, agg_ref, w1_ref, b1_ref, a1_ref, gi_ref, bi_ref,
             w2_ref, b2_ref, a2_ref, go_ref, bo_ref, o_ref):
        m = h_ref[...] + agg_ref[0, :_N] + agg_ref[1, :_N]
        t = jnp.dot(m, w1_ref[...],
                    preferred_element_type=jnp.float32) + b1_ref[...]
        t = jnp.where(t >= 0, t, a1_ref[0, 0] * t)
        mu = jnp.mean(t, axis=0, keepdims=True)
        var = jnp.mean((t - mu) ** 2, axis=0, keepdims=True)
        t = gi_ref[...] * (t - mu) / jnp.sqrt(var + 1e-5) + bi_ref[...]
        t = jnp.dot(t, w2_ref[...],
                    preferred_element_type=jnp.float32) + b2_ref[...]
        t = jnp.where(t >= 0, t, a2_ref[0, 0] * t)
        mu2 = jnp.mean(t, axis=0, keepdims=True)
        var2 = jnp.mean((t - mu2) ** 2, axis=0, keepdims=True)
        o_ref[...] = (go_ref[...] * (t - mu2) / jnp.sqrt(var2 + 1e-5)
                      + bo_ref[...])

    return pl.pallas_call(
        body,
        out_shape=jax.ShapeDtypeStruct((_N, _H), jnp.float32),
    )(h, agg2, W1, b1.reshape(1, _H), a1.reshape(1, 1), gin.reshape(1, _H),
      bein.reshape(1, _H), W2, b2.reshape(1, _H), a2.reshape(1, 1),
      gout.reshape(1, _H), beout.reshape(1, _H))


def _tc_final(h, batch2d, bn_g, bn_b, fc_W, fc_b):
    def body(h_ref, b_ref, g_ref, bb_ref, w_ref, fb_ref, o_ref):
        gids = lax.broadcasted_iota(jnp.int32, (_G, _N), 0)
        oh = (b_ref[...] == gids).astype(jnp.float32)
        pooled = jnp.dot(oh, h_ref[...], preferred_element_type=jnp.float32)
        mu = jnp.mean(pooled, axis=0, keepdims=True)
        var = jnp.mean((pooled - mu) ** 2, axis=0, keepdims=True)
        z = g_ref[...] * (pooled - mu) / jnp.sqrt(var + 1e-5) + bb_ref[...]
        o_ref[...] = jnp.dot(z, w_ref[...],
                             preferred_element_type=jnp.float32) + fb_ref[...]

    lat = fc_W.shape[1]
    return pl.pallas_call(
        body,
        out_shape=jax.ShapeDtypeStruct((_G, lat), jnp.float32),
    )(h, batch2d, bn_g.reshape(1, _H), bn_b.reshape(1, _H), fc_W,
      fc_b.reshape(1, lat))


def kernel(x, edge_index, batch, fc0_W, fc0_b, W1, b1, a1, g_in, be_in,
           W2, b2, a2, g_out, be_out, bn_g, bn_b, fc_W, fc_b):
    src = edge_index[0]
    dst = edge_index[1]
    pad = _EPAD - _E
    srcr = jnp.concatenate(
        [src, jnp.full((pad,), _N, src.dtype)]).reshape(_NW, _NCH, _CW)
    dstr = jnp.concatenate(
        [dst, jnp.zeros((pad,), dst.dtype)]).reshape(_NW, _NCH, _CW)
    zrows = jnp.zeros((_RPT, _H), jnp.float32)

    h = _tc_fc0(x, fc0_W, fc0_b)
    for i in range(_NL):
        hp = jnp.concatenate([h, jnp.zeros((8, _H), jnp.float32)], axis=0)
        agg2 = _sc_agg(hp, srcr, dstr, zrows)
        h = _tc_layer(h, agg2, W1[i], b1[i], a1[i].reshape(1, 1),
                      g_in[i], be_in[i], W2[i], b2[i], a2[i].reshape(1, 1),
                      g_out[i], be_out[i])
    return _tc_final(h, batch.reshape(1, _N), bn_g, bn_b, fc_W, fc_b)


# speedup vs baseline: 1.3689x; 1.0010x over previous
import functools

import jax
import jax.numpy as jnp
from jax import lax
from jax.experimental import pallas as pl
from jax.experimental.pallas import tpu as pltpu
from jax.experimental.pallas import tpu_sc as plsc

_N = 10000
_E = 320000
_H = 128
_G = 64
_NL = 3
_NC = 2
_NS = 16
_NW = _NC * _NS
_CW = 128
_NCH = 80
_EPT = _CW * _NCH
_EPAD = _EPT * _NW
_APAD = 10112
_RPT = _APAD // _NS
_HCH = _NCH // 2


def _sc_agg(h_pad, srcr, dstr, zrows):
    mesh = plsc.VectorSubcoreMesh(core_axis_name="c", subcore_axis_name="s")

    @functools.partial(
        pl.kernel,
        out_type=jax.ShapeDtypeStruct((_NC, _APAD, _H), jnp.float32),
        mesh=mesh,
        scratch_types=[
            pltpu.VMEM((_HCH, _CW), jnp.int32),
            pltpu.VMEM((_HCH, _CW), jnp.int32),
            pltpu.VMEM((_CW, _H), jnp.float32),
            pltpu.VMEM((_CW, _H), jnp.float32),
            pltpu.VMEM_SHARED((_APAD, _H), jnp.float32),
            pltpu.SemaphoreType.DMA,
        ],
    )
    def agg(h_hbm, src_hbm, dst_hbm, z_hbm, out_hbm, src_v, dst_v, rows_a,
            rows_b, acc, sem):
        c = lax.axis_index("c")
        s = lax.axis_index("s")
        wid = s * _NC + c
        r0 = s * _RPT
        pltpu.sync_copy(z_hbm, acc.at[pl.ds(r0, _RPT)])
        plsc.subcore_barrier()

        for half in range(2):
            pltpu.sync_copy(src_hbm.at[wid, pl.ds(half * _HCH, _HCH)], src_v)
            pltpu.sync_copy(dst_hbm.at[wid, pl.ds(half * _HCH, _HCH)], dst_v)
            pltpu.async_copy(h_hbm.at[src_v.at[0]], rows_a, sem).wait()

            def step(jo, carry):
                j = 2 * jo
                cp_b = pltpu.async_copy(h_hbm.at[src_v.at[j + 1]], rows_b,
                                        sem)
                pltpu.sync_copy(rows_a, acc.at[dst_v.at[j]], add=True)
                cp_b.wait()
                j2 = jnp.minimum(j + 2, _HCH - 1)
                cp_a = pltpu.async_copy(h_hbm.at[src_v.at[j2]], rows_a, sem)
                pltpu.sync_copy(rows_b, acc.at[dst_v.at[j + 1]], add=True)
                cp_a.wait()
                return carry

            lax.fori_loop(0, _HCH // 2, step, 0)
        plsc.subcore_barrier()
        pltpu.sync_copy(acc.at[pl.ds(r0, _RPT)],
                        out_hbm.at[c, pl.ds(r0, _RPT)])

    return agg(h_pad, srcr, dstr, zrows)


def _tc_fc0(x, W, b):
    def body(x_ref, w_ref, b_ref, o_ref):
        o_ref[...] = jnp.dot(x_ref[...], w_ref[...],
                             preferred_element_type=jnp.float32) + b_ref[...]

    return pl.pallas_call(
        body,
        out_shape=jax.ShapeDtypeStruct((_N, _H), jnp.float32),
    )(x, W, b.reshape(1, _H))


def _tc_layer(h, agg2, W1, b1, a1, gin, bein, W2, b2, a2, gout, beout):
    def body(h_ref, agg_ref, w1_ref, b1_ref, a1_ref, gi_ref, bi_ref,
             w2_ref, b2_ref, a2_ref, go_ref, bo_ref, o_ref):
        m = h_ref[...] + agg_ref[0, :_N] + agg_ref[1, :_N]
        t = jnp.dot(m, w1_ref[...],
                    preferred_element_type=jnp.float32) + b1_ref[...]
        t = jnp.where(t >= 0, t, a1_ref[0, 0] * t)
        mu = jnp.mean(t, axis=0, keepdims=True)
        var = jnp.mean((t - mu) ** 2, axis=0, keepdims=True)
        t = gi_ref[...] * (t - mu) / jnp.sqrt(var + 1e-5) + bi_ref[...]
        t = jnp.dot(t, w2_ref[...],
                    preferred_element_type=jnp.float32) + b2_ref[...]
        t = jnp.where(t >= 0, t, a2_ref[0, 0] * t)
        mu2 = jnp.mean(t, axis=0, keepdims=True)
        var2 = jnp.mean((t - mu2) ** 2, axis=0, keepdims=True)
        o_ref[...] = (go_ref[...] * (t - mu2) / jnp.sqrt(var2 + 1e-5)
                      + bo_ref[...])

    return pl.pallas_call(
        body,
        out_shape=jax.ShapeDtypeStruct((_N, _H), jnp.float32),
    )(h, agg2, W1, b1.reshape(1, _H), a1.reshape(1, 1), gin.reshape(1, _H),
      bein.reshape(1, _H), W2, b2.reshape(1, _H), a2.reshape(1, 1),
      gout.reshape(1, _H), beout.reshape(1, _H))


def _tc_last(h, agg2, W1, b1, a1, gin, bein, W2, b2, a2, gout, beout,
             batch2d, bn_g, bn_b, fc_W, fc_b):
    def body(h_ref, agg_ref, w1_ref, b1_ref, a1_ref, gi_ref, bi_ref,
             w2_ref, b2_ref, a2_ref, go_ref, bo_ref, b_ref, g_ref, bb_ref,
             w_ref, fb_ref, o_ref):
        m = h_ref[...] + agg_ref[0, :_N] + agg_ref[1, :_N]
        t = jnp.dot(m, w1_ref[...],
                    preferred_element_type=jnp.float32) + b1_ref[...]
        t = jnp.where(t >= 0, t, a1_ref[0, 0] * t)
        mu = jnp.mean(t, axis=0, keepdims=True)
        var = jnp.mean((t - mu) ** 2, axis=0, keepdims=True)
        t = gi_ref[...] * (t - mu) / jnp.sqrt(var + 1e-5) + bi_ref[...]
        t = jnp.dot(t, w2_ref[...],
                    preferred_element_type=jnp.float32) + b2_ref[...]
        t = jnp.where(t >= 0, t, a2_ref[0, 0] * t)
        mu2 = jnp.mean(t, axis=0, keepdims=True)
        var2 = jnp.mean((t - mu2) ** 2, axis=0, keepdims=True)
        hl = (go_ref[...] * (t - mu2) / jnp.sqrt(var2 + 1e-5)
              + bo_ref[...])
        gids = lax.broadcasted_iota(jnp.int32, (_G, _N), 0)
        oh = (b_ref[...] == gids).astype(jnp.float32)
        pooled = jnp.dot(oh, hl, preferred_element_type=jnp.float32)
        mu3 = jnp.mean(pooled, axis=0, keepdims=True)
        var3 = jnp.mean((pooled - mu3) ** 2, axis=0, keepdims=True)
        z = g_ref[...] * (pooled - mu3) / jnp.sqrt(var3 + 1e-5) + bb_ref[...]
        o_ref[...] = jnp.dot(z, w_ref[...],
                             preferred_element_type=jnp.float32) + fb_ref[...]

    lat = fc_W.shape[1]
    return pl.pallas_call(
        body,
        out_shape=jax.ShapeDtypeStruct((_G, lat), jnp.float32),
    )(h, agg2, W1, b1.reshape(1, _H), a1.reshape(1, 1), gin.reshape(1, _H),
      bein.reshape(1, _H), W2, b2.reshape(1, _H), a2.reshape(1, 1),
      gout.reshape(1, _H), beout.reshape(1, _H), batch2d,
      bn_g.reshape(1, _H), bn_b.reshape(1, _H), fc_W, fc_b.reshape(1, lat))


def kernel(x, edge_index, batch, fc0_W, fc0_b, W1, b1, a1, g_in, be_in,
           W2, b2, a2, g_out, be_out, bn_g, bn_b, fc_W, fc_b):
    src = edge_index[0]
    dst = edge_index[1]
    pad = _EPAD - _E
    srcr = jnp.concatenate(
        [src, jnp.full((pad,), _N, src.dtype)]).reshape(_NW, _NCH, _CW)
    dstr = jnp.concatenate(
        [dst, jnp.zeros((pad,), dst.dtype)]).reshape(_NW, _NCH, _CW)
    zrows = jnp.zeros((_RPT, _H), jnp.float32)

    h = _tc_fc0(x, fc0_W, fc0_b)
    for i in range(_NL - 1):
        hp = jnp.concatenate([h, jnp.zeros((8, _H), jnp.float32)], axis=0)
        agg2 = _sc_agg(hp, srcr, dstr, zrows)
        h = _tc_layer(h, agg2, W1[i], b1[i], a1[i].reshape(1, 1),
                      g_in[i], be_in[i], W2[i], b2[i], a2[i].reshape(1, 1),
                      g_out[i], be_out[i])
    i = _NL - 1
    hp = jnp.concatenate([h, jnp.zeros((8, _H), jnp.float32)], axis=0)
    agg2 = _sc_agg(hp, srcr, dstr, zrows)
    return _tc_last(h, agg2, W1[i], b1[i], a1[i].reshape(1, 1),
                    g_in[i], be_in[i], W2[i], b2[i], a2[i].reshape(1, 1),
                    g_out[i], be_out[i], batch.reshape(1, _N),
                    bn_g, bn_b, fc_W, fc_b)
